# Initial kernel scaffold; baseline (speedup 1.0000x reference)
import jax
import jax.numpy as jnp
from jax.experimental import pallas as pl


def kernel(stacked_raw_attributes, blocks_mask):
    raise NotImplementedError("write your pallas kernel here")


# speedup vs baseline: 1.7641x; 1.7641x over previous
import jax
import jax.numpy as jnp
import numpy as np
from jax import lax
from jax.experimental import pallas as pl
from jax.experimental.pallas import tpu as pltpu

HASH_MOD = 2**31 - 1
NUM_ATTRS = 100
R = 128
C = 128
N = R * C

_W = tuple(
    int(v)
    for v in (
        np.random.default_rng(1234)
        .integers(1, HASH_MOD, size=(NUM_ATTRS,), dtype=np.int64)
        .astype(np.int32)
        | 1
    )
)


def _row_iota():
    return lax.broadcasted_iota(jnp.int32, (R, C), 0)


def _lane_iota():
    return lax.broadcasted_iota(jnp.int32, (R, C), 1)


def _bit_zero(j):
    if j >= N:
        return jnp.full((R, C), True)
    if j < C:
        return (_lane_iota() & j) == 0
    return (_row_iota() & (j // C)) == 0


def _partner(x, j):
    if j < C:
        bit = _bit_zero(j)
        return jnp.where(bit, jnp.roll(x, -j, axis=1), jnp.roll(x, j, axis=1))
    jr = j // C
    bit = _bit_zero(j)
    return jnp.where(bit, jnp.roll(x, -jr, axis=0), jnp.roll(x, jr, axis=0))


def _cx_pair(h, v, j, k):
    hp = _partner(h, j)
    vp = _partner(v, j)
    tm = _bit_zero(j) == _bit_zero(k)
    keep = (h == hp) | ((h < hp) == tm)
    return jnp.where(keep, h, hp), jnp.where(keep, v, vp)


def _cx_val(h, j, k):
    hp = _partner(h, j)
    tm = _bit_zero(j) == _bit_zero(k)
    keep = (h == hp) | ((h < hp) == tm)
    return jnp.where(keep, h, hp)


def _prefix_incl(x):
    lane = _lane_iota()
    for s in (1, 2, 4, 8, 16, 32, 64):
        x = x + jnp.where(lane >= s, jnp.roll(x, s, axis=1), 0)
    rowtot = jax.lax.broadcast_in_dim(x[:, C - 1], (R, C), (0,))
    row = _row_iota()
    for s in (1, 2, 4, 8, 16, 32, 64):
        rowtot = rowtot + jnp.where(row >= s, jnp.roll(rowtot, s, axis=0), 0)
    return x + jnp.where(row >= 1, jnp.roll(rowtot, 1, axis=0), 0)


def _body(xt_ref, mask_ref, out_ref):
    acc = jnp.zeros((R, C), jnp.int32)
    bias = jnp.int32(0)
    for a in range(NUM_ATTRS):
        wm = jnp.where(mask_ref[0, a] != 0, jnp.int32(_W[a]), jnp.int32(0))
        acc = acc + xt_ref[a] * wm
        bias = bias + wm
    s = acc + bias
    h = s % HASH_MOD
    h = jnp.where(h < 0, h + HASH_MOD, h)

    idx = _row_iota() * C + _lane_iota()
    k = 2
    while k <= N:
        j = k // 2
        while j >= 1:
            h, idx = _cx_pair(h, idx, j, k)
            j //= 2
        k *= 2

    p1 = jnp.roll(h, 1, axis=1)
    prev = jnp.where(_lane_iota() == 0, jnp.roll(p1, 1, axis=0), p1)
    e0 = (_row_iota() == 0) & (_lane_iota() == 0)
    f = (e0 | (h != prev)).astype(jnp.int32)
    rank = _prefix_incl(f) - 1

    p = (idx << 15) | rank
    k = 2
    while k <= N:
        j = k // 2
        while j >= 1:
            p = _cx_val(p, j, k)
            j //= 2
        k *= 2
    out_ref[...] = p & 0x7FFF


def _ranks_2d(xt, mask_i32, interpret=False):
    return pl.pallas_call(
        _body,
        out_shape=jax.ShapeDtypeStruct((R, C), jnp.int32),
        in_specs=[
            pl.BlockSpec(memory_space=pltpu.VMEM),
            pl.BlockSpec(memory_space=pltpu.SMEM),
        ],
        out_specs=pl.BlockSpec(memory_space=pltpu.VMEM),
        interpret=interpret,
    )(xt, mask_i32)


def kernel(stacked_raw_attributes, blocks_mask):
    key = jax.random.key(42)
    k_idx, _k_branch, _k_split, _k_collide = jax.random.split(key, 4)
    n_blocks = blocks_mask.shape[0]
    random_index = jax.random.randint(k_idx, (), 0, n_blocks)
    chosen = blocks_mask[random_index]
    mask_i32 = chosen.astype(jnp.int32).reshape(1, NUM_ATTRS)
    xt = stacked_raw_attributes.T.reshape(NUM_ATTRS, R, C)
    return _ranks_2d(xt, mask_i32).reshape(-1)
